# initial kernel scaffold (unmeasured)
import jax
import jax.numpy as jnp
from jax import lax
from jax.experimental import pallas as pl
from jax.experimental.pallas import tpu as pltpu

N_DEV = 4
N_EXP = 16
E_LOCAL = N_EXP // N_DEV
N_TOK = 1024
D_IN = 512
D_OUT = 1024
CHUNK = N_TOK // N_DEV


def kernel(x, router_W, route_idx, expert_W):
    def body(x_ref, rw_ref, idx_ref, ew_ref, out_ref,
             partial_ref, comm_ref, send_sems, recv_sems):
        p = lax.axis_index("i")
        left = lax.rem(p + N_DEV - 1, N_DEV)
        right = lax.rem(p + 1, N_DEV)

        bar = pltpu.get_barrier_semaphore()
        pl.semaphore_signal(bar, inc=1, device_id=(left,),
                            device_id_type=pl.DeviceIdType.MESH)
        pl.semaphore_wait(bar, 1)

        xv = x_ref[...]
        scores = jnp.dot(xv, rw_ref[...], preferred_element_type=jnp.float32)
        smax = jnp.max(scores, axis=-1, keepdims=True)
        ex = jnp.exp(scores - smax)
        idx = idx_ref[...]
        iota = lax.broadcasted_iota(jnp.int32, (N_TOK, N_EXP), 1)
        routed = (idx[:, 0:1] == iota) | (idx[:, 1:2] == iota)
        exm = jnp.where(routed, ex, 0.0)
        gates = exm / jnp.sum(exm, axis=-1, keepdims=True)
        gl = lax.dynamic_slice_in_dim(gates, p * E_LOCAL, E_LOCAL, axis=1)

        partial = jnp.zeros((N_TOK, D_OUT), jnp.float32)
        for e in range(E_LOCAL):
            xw = xv * gl[:, e:e + 1]
            partial = partial + jnp.dot(xw, ew_ref[e],
                                        preferred_element_type=jnp.float32)
        partial_ref[...] = partial

        for s in range(N_DEV - 1):
            c = lax.rem(p + (N_DEV - 1 - s) + N_DEV, N_DEV)
            row0 = c * CHUNK
            if s == 0:
                src = partial_ref.at[pl.ds(row0, CHUNK), :]
            else:
                comm_ref[s - 1] = (comm_ref[s - 1]
                                   + partial_ref[pl.ds(row0, CHUNK), :])
                src = comm_ref.at[s - 1]
            rdma = pltpu.make_async_remote_copy(
                src_ref=src,
                dst_ref=comm_ref.at[s],
                send_sem=send_sems.at[s],
                recv_sem=recv_sems.at[s],
                device_id=(right,),
                device_id_type=pl.DeviceIdType.MESH,
            )
            rdma.start()
            rdma.wait()

        out_ref[...] = comm_ref[N_DEV - 2] + partial_ref[pl.ds(p * CHUNK, CHUNK), :]

    return pl.pallas_call(
        body,
        out_shape=jax.ShapeDtypeStruct((CHUNK, D_OUT), jnp.float32),
        in_specs=[
            pl.BlockSpec(memory_space=pltpu.VMEM),
            pl.BlockSpec(memory_space=pltpu.VMEM),
            pl.BlockSpec(memory_space=pltpu.VMEM),
            pl.BlockSpec(memory_space=pltpu.VMEM),
        ],
        out_specs=pl.BlockSpec(memory_space=pltpu.VMEM),
        scratch_shapes=[
            pltpu.VMEM((N_TOK, D_OUT), jnp.float32),
            pltpu.VMEM((N_DEV - 1, CHUNK, D_OUT), jnp.float32),
            pltpu.SemaphoreType.DMA((N_DEV - 1,)),
            pltpu.SemaphoreType.DMA((N_DEV - 1,)),
        ],
        compiler_params=pltpu.CompilerParams(collective_id=0),
    )(x, router_W, route_idx, expert_W)


# baseline (device time: 56641 ns/iter reference)
import jax
import jax.numpy as jnp
from jax import lax
from jax.experimental import pallas as pl
from jax.experimental.pallas import tpu as pltpu

N_DEV = 4
N_EXP = 16
E_LOCAL = N_EXP // N_DEV
N_TOK = 1024
D_IN = 512
D_OUT = 1024
CHUNK = N_TOK // N_DEV


def kernel(x, router_W, route_idx, expert_W):
    def body(x_ref, rw_ref, idx_ref, ew_ref, out_ref,
             partial_ref, comm_ref, send_sems, recv_sems):
        p = lax.axis_index("i")
        left = lax.rem(p + N_DEV - 1, N_DEV)
        right = lax.rem(p + 1, N_DEV)

        bar = pltpu.get_barrier_semaphore()
        pl.semaphore_signal(bar, inc=1, device_id=(left,),
                            device_id_type=pl.DeviceIdType.MESH)
        pl.semaphore_wait(bar, 1)

        xv = x_ref[...]
        scores = jnp.dot(xv, rw_ref[...], preferred_element_type=jnp.float32)
        smax = jnp.max(scores, axis=-1, keepdims=True)
        ex = jnp.exp(scores - smax)
        idx = idx_ref[...]
        iota = lax.broadcasted_iota(jnp.int32, (N_TOK, N_EXP), 1)
        routed = (idx[:, 0:1] == iota) | (idx[:, 1:2] == iota)
        exm = jnp.where(routed, ex, 0.0)
        denom = jnp.sum(exm, axis=-1, keepdims=True)

        partial = jnp.zeros((N_TOK, D_OUT), jnp.float32)
        for e in range(E_LOCAL):
            ge = p * E_LOCAL + e
            num = jnp.sum(jnp.where(iota == ge, exm, 0.0),
                          axis=-1, keepdims=True)
            xw = xv * (num / denom)
            partial = partial + jnp.dot(xw, ew_ref[e],
                                        preferred_element_type=jnp.float32)
        partial_ref[...] = partial

        for s in range(N_DEV - 1):
            c = lax.rem(p + (N_DEV - 1 - s) + N_DEV, N_DEV)
            row0 = c * CHUNK
            if s == 0:
                src = partial_ref.at[pl.ds(row0, CHUNK), :]
            else:
                comm_ref[s - 1] = (comm_ref[s - 1]
                                   + partial_ref[pl.ds(row0, CHUNK), :])
                src = comm_ref.at[s - 1]
            rdma = pltpu.make_async_remote_copy(
                src_ref=src,
                dst_ref=comm_ref.at[s],
                send_sem=send_sems.at[s],
                recv_sem=recv_sems.at[s],
                device_id=(right,),
                device_id_type=pl.DeviceIdType.MESH,
            )
            rdma.start()
            rdma.wait()

        out_ref[...] = comm_ref[N_DEV - 2] + partial_ref[pl.ds(p * CHUNK, CHUNK), :]

    return pl.pallas_call(
        body,
        out_shape=jax.ShapeDtypeStruct((CHUNK, D_OUT), jnp.float32),
        in_specs=[
            pl.BlockSpec(memory_space=pltpu.VMEM),
            pl.BlockSpec(memory_space=pltpu.VMEM),
            pl.BlockSpec(memory_space=pltpu.VMEM),
            pl.BlockSpec(memory_space=pltpu.VMEM),
        ],
        out_specs=pl.BlockSpec(memory_space=pltpu.VMEM),
        scratch_shapes=[
            pltpu.VMEM((N_TOK, D_OUT), jnp.float32),
            pltpu.VMEM((N_DEV - 1, CHUNK, D_OUT), jnp.float32),
            pltpu.SemaphoreType.DMA((N_DEV - 1,)),
            pltpu.SemaphoreType.DMA((N_DEV - 1,)),
        ],
        compiler_params=pltpu.CompilerParams(collective_id=0),
    )(x, router_W, route_idx, expert_W)


# device time: 54375 ns/iter; 1.0417x vs baseline; 1.0417x over previous
import jax
import jax.numpy as jnp
from jax import lax
from jax.experimental import pallas as pl
from jax.experimental.pallas import tpu as pltpu

N_DEV = 4
N_EXP = 16
E_LOCAL = N_EXP // N_DEV
N_TOK = 1024
D_IN = 512
D_OUT = 1024
CHUNK = N_TOK // N_DEV


def kernel(x, router_W, route_idx, expert_W):
    def body(x_ref, rw_ref, idx_ref, ew_ref, out_ref,
             xw_ref, sbuf_ref, comm_ref, send_sems, recv_sems):
        p = lax.axis_index("i")
        left = lax.rem(p + N_DEV - 1, N_DEV)
        right = lax.rem(p + 1, N_DEV)

        bar = pltpu.get_barrier_semaphore()
        pl.semaphore_signal(bar, inc=1, device_id=(left,),
                            device_id_type=pl.DeviceIdType.MESH)
        pl.semaphore_wait(bar, 1)

        xv = x_ref[...]
        scores = jnp.dot(xv, rw_ref[...], preferred_element_type=jnp.float32)
        smax = jnp.max(scores, axis=-1, keepdims=True)
        ex = jnp.exp(scores - smax)
        idx = idx_ref[...]
        iota = lax.broadcasted_iota(jnp.int32, (N_TOK, N_EXP), 1)
        routed = (idx[:, 0:1] == iota) | (idx[:, 1:2] == iota)
        exm = jnp.where(routed, ex, 0.0)
        denom = jnp.sum(exm, axis=-1, keepdims=True)
        for e in range(E_LOCAL):
            ge = p * E_LOCAL + e
            num = jnp.sum(jnp.where(iota == ge, exm, 0.0),
                          axis=-1, keepdims=True)
            xw_ref[e] = xv * (num / denom)

        def chunk_partial(row0):
            acc = jnp.zeros((CHUNK, D_OUT), jnp.float32)
            for e in range(E_LOCAL):
                acc = acc + jnp.dot(xw_ref[e, pl.ds(row0, CHUNK), :],
                                    ew_ref[e],
                                    preferred_element_type=jnp.float32)
            return acc

        sbuf_ref[...] = chunk_partial(lax.rem(p + 3, N_DEV) * CHUNK)
        rdma = pltpu.make_async_remote_copy(
            src_ref=sbuf_ref,
            dst_ref=comm_ref.at[0],
            send_sem=send_sems.at[0],
            recv_sem=recv_sems.at[0],
            device_id=(right,),
            device_id_type=pl.DeviceIdType.MESH,
        )
        rdma.start()
        for s in range(1, N_DEV - 1):
            pc = chunk_partial(lax.rem(p + 3 - s + N_DEV, N_DEV) * CHUNK)
            rdma.wait()
            comm_ref[s - 1] = comm_ref[s - 1] + pc
            rdma = pltpu.make_async_remote_copy(
                src_ref=comm_ref.at[s - 1],
                dst_ref=comm_ref.at[s],
                send_sem=send_sems.at[s],
                recv_sem=recv_sems.at[s],
                device_id=(right,),
                device_id_type=pl.DeviceIdType.MESH,
            )
            rdma.start()
        pc_own = chunk_partial(p * CHUNK)
        rdma.wait()
        out_ref[...] = comm_ref[N_DEV - 2] + pc_own

    return pl.pallas_call(
        body,
        out_shape=jax.ShapeDtypeStruct((CHUNK, D_OUT), jnp.float32),
        in_specs=[
            pl.BlockSpec(memory_space=pltpu.VMEM),
            pl.BlockSpec(memory_space=pltpu.VMEM),
            pl.BlockSpec(memory_space=pltpu.VMEM),
            pl.BlockSpec(memory_space=pltpu.VMEM),
        ],
        out_specs=pl.BlockSpec(memory_space=pltpu.VMEM),
        scratch_shapes=[
            pltpu.VMEM((E_LOCAL, N_TOK, D_IN), jnp.float32),
            pltpu.VMEM((CHUNK, D_OUT), jnp.float32),
            pltpu.VMEM((N_DEV - 1, CHUNK, D_OUT), jnp.float32),
            pltpu.SemaphoreType.DMA((N_DEV - 1,)),
            pltpu.SemaphoreType.DMA((N_DEV - 1,)),
        ],
        compiler_params=pltpu.CompilerParams(collective_id=0),
    )(x, router_W, route_idx, expert_W)


# device time: 23236 ns/iter; 2.4376x vs baseline; 2.3401x over previous
import jax
import jax.numpy as jnp
from jax import lax
from jax.experimental import pallas as pl
from jax.experimental.pallas import tpu as pltpu

N_DEV = 4
N_EXP = 16
E_LOCAL = N_EXP // N_DEV
N_TOK = 1024
D_IN = 512
D_OUT = 1024
CHUNK = N_TOK // N_DEV
HALF = D_OUT // 2
QUART = D_OUT // 4

F32 = jnp.float32
BF16 = jnp.bfloat16


def kernel(x, router_W, route_idx, expert_W):
    def body(x_ref, rw_ref, idx_ref, ew_ref, out_ref,
             gates_ref, sbq, sbh, rvq, rvh, ssem, rsem):
        p = lax.axis_index("i")
        left = lax.rem(p + N_DEV - 1, N_DEV)
        right = lax.rem(p + 1, N_DEV)

        bar = pltpu.get_barrier_semaphore()
        for nbr in (left, right):
            pl.semaphore_signal(bar, inc=1, device_id=(nbr,),
                                device_id_type=pl.DeviceIdType.MESH)

        scores = jnp.dot(x_ref[...], rw_ref[...], preferred_element_type=F32)
        smax = jnp.max(scores, axis=-1, keepdims=True)
        ex = jnp.exp(scores - smax)
        idx = idx_ref[...]
        iota = lax.broadcasted_iota(jnp.int32, (N_TOK, N_EXP), 1)
        routed = (idx[:, 0:1] == iota) | (idx[:, 1:2] == iota)
        exm = jnp.where(routed, ex, 0.0)
        gates_ref[...] = exm / jnp.sum(exm, axis=-1, keepdims=True)

        def partial_block(chunk_id, col0, width):
            row0 = chunk_id * CHUNK
            xr = x_ref[pl.ds(row0, CHUNK), :]
            gr = gates_ref[pl.ds(row0, CHUNK), :]
            iota_c = lax.broadcasted_iota(jnp.int32, (CHUNK, N_EXP), 1)
            acc = jnp.zeros((CHUNK, width), F32)
            for e in range(E_LOCAL):
                g = jnp.sum(jnp.where(iota_c == p * E_LOCAL + e, gr, 0.0),
                            axis=-1, keepdims=True)
                acc = acc + g * jnp.dot(xr, ew_ref[e, :, col0:col0 + width],
                                        preferred_element_type=F32)
            return acc

        def copy(src, dst, k, dev):
            return pltpu.make_async_remote_copy(
                src_ref=src, dst_ref=dst, send_sem=ssem.at[k],
                recv_sem=rsem.at[k], device_id=(dev,),
                device_id_type=pl.DeviceIdType.MESH)

        c_right = lax.rem(p + 1, N_DEV)
        c_diag = lax.rem(p + 2, N_DEV)
        c_left = lax.rem(p + 3, N_DEV)

        leg1 = []
        for k, (col0, dev) in enumerate([(0, right), (QUART, right),
                                         (HALF, left), (HALF + QUART, left)]):
            sbq[k] = partial_block(c_diag, col0, QUART).astype(BF16)
            if k == 0:
                pl.semaphore_wait(bar, 2)
            d = copy(sbq.at[k], rvq.at[k], k, dev)
            d.start()
            leg1.append(d)

        parts = [partial_block(c_right, 0, QUART),
                 partial_block(c_right, QUART, QUART),
                 partial_block(c_left, HALF, QUART),
                 partial_block(c_left, HALF + QUART, QUART)]

        leg2 = []
        for k, dev in enumerate([right, right, left, left]):
            leg1[k].wait()
            rvq[k] = (rvq[k].astype(F32) + parts[k]).astype(BF16)
            d = copy(rvq.at[k], rvq.at[4 + k], 4 + k, dev)
            d.start()
            leg2.append(d)

        sbh[0] = partial_block(c_left, 0, HALF).astype(BF16)
        d8 = copy(sbh.at[0], rvh.at[0], 8, left)
        d8.start()
        sbh[1] = partial_block(c_right, HALF, HALF).astype(BF16)
        d9 = copy(sbh.at[1], rvh.at[1], 9, right)
        d9.start()

        own0 = partial_block(p, 0, HALF)
        own1 = partial_block(p, HALF, HALF)

        leg2[0].wait()
        leg2[1].wait()
        d8.wait()
        out_ref[:, :HALF] = (
            own0 + rvh[0].astype(F32)
            + jnp.concatenate([rvq[4].astype(F32), rvq[5].astype(F32)], axis=1))
        leg2[2].wait()
        leg2[3].wait()
        d9.wait()
        out_ref[:, HALF:] = (
            own1 + rvh[1].astype(F32)
            + jnp.concatenate([rvq[6].astype(F32), rvq[7].astype(F32)], axis=1))

    return pl.pallas_call(
        body,
        out_shape=jax.ShapeDtypeStruct((CHUNK, D_OUT), F32),
        in_specs=[
            pl.BlockSpec(memory_space=pltpu.VMEM),
            pl.BlockSpec(memory_space=pltpu.VMEM),
            pl.BlockSpec(memory_space=pltpu.VMEM),
            pl.BlockSpec(memory_space=pltpu.VMEM),
        ],
        out_specs=pl.BlockSpec(memory_space=pltpu.VMEM),
        scratch_shapes=[
            pltpu.VMEM((N_TOK, N_EXP), F32),
            pltpu.VMEM((4, CHUNK, QUART), BF16),
            pltpu.VMEM((2, CHUNK, HALF), BF16),
            pltpu.VMEM((8, CHUNK, QUART), BF16),
            pltpu.VMEM((2, CHUNK, HALF), BF16),
            pltpu.SemaphoreType.DMA((10,)),
            pltpu.SemaphoreType.DMA((10,)),
        ],
        compiler_params=pltpu.CompilerParams(collective_id=0),
    )(x, router_W, route_idx, expert_W)
